# baseline (speedup 1.0000x reference)
import functools

import jax
import jax.numpy as jnp
from jax import lax
from jax.experimental import pallas as pl
from jax.experimental.pallas import tpu as pltpu
from jax.experimental.pallas import tpu_sc as plsc

N = 10000
E = 320000
NUM_RELS = 90
SUBMAT = 32

NC = 2
NS = 16
NW = NC * NS

NPT = 313
NLAST = N - (NW - 1) * NPT

CS = 128
PCH = 20
NPH = E // (PCH * CS)

CAP = 12288
GC = 128

TN = 2000
NT = N // TN


def _y_body(w_ref, x_ref, y_ref):
    w = w_ref[...]
    wcat = jnp.concatenate([w, w, w, w], axis=1)
    ri = lax.broadcasted_iota(jnp.int32, (128, 128), 0)
    ci = lax.broadcasted_iota(jnp.int32, (128, 128), 1)
    wbd = jnp.where((ri // SUBMAT) == (ci // SUBMAT), wcat, 0.0)
    y_ref[...] = jnp.dot(x_ref[...], wbd, preferred_element_type=jnp.float32)


def _make_y(x, wr):
    return pl.pallas_call(
        _y_body,
        grid=(NT, NUM_RELS),
        in_specs=[
            pl.BlockSpec((128, 32), lambda nt, r: (r, 0)),
            pl.BlockSpec((TN, 128), lambda nt, r: (nt, 0)),
        ],
        out_specs=pl.BlockSpec((TN, 128), lambda nt, r: (r * NT + nt, 0)),
        out_shape=jax.ShapeDtypeStruct((NUM_RELS * N, 128), jnp.float32),
    )(wr, x)


def _sc_body(y_hbm, src_hbm, dst_hbm, typ_hbm, out_hbm,
             sad, sas, sat, sbd, sbs, sbt, idx_c, dloc_c, h2d,
             rows_a, rows_b, sem_sa, sem_sb, sem_a, sem_b):
    cid = lax.axis_index("c")
    sid = lax.axis_index("s")
    wid = cid * NS + sid
    lo = wid * NPT

    zf = jnp.zeros((16,), jnp.float32)
    def _z(r, c):
        for k in range(8):
            h2d.at[r][pl.ds(k * 16, 16)] = zf
        return c
    lax.fori_loop(0, NPT + 1, _z, 0)

    bufs_a = (sad, sas, sat)
    bufs_b = (sbd, sbs, sbt)

    def _stage(p, bufs, sem):
        d, s, t = bufs
        pltpu.async_copy(dst_hbm.at[p], d, sem)
        pltpu.async_copy(src_hbm.at[p], s, sem)
        pltpu.async_copy(typ_hbm.at[p], t, sem)

    def _stage_wait(bufs, sem):
        d, s, t = bufs
        pltpu.make_async_copy(dst_hbm.at[0], d, sem).wait()
        pltpu.make_async_copy(src_hbm.at[0], s, sem).wait()
        pltpu.make_async_copy(typ_hbm.at[0], t, sem).wait()

    def _scan(bufs, ptr):
        d, s, t = bufs
        def _row(j, ptr):
            for k in range(8):
                sl = pl.ds(k * 16, 16)
                dv = d.at[j][sl]
                m = (dv >= lo) & (dv < lo + NPT)
                idx = t.at[j][sl] * N + s.at[j][sl]
                pos = jnp.broadcast_to(ptr - 1, (16,)) + plsc.cumsum(m.astype(jnp.int32))
                plsc.store_scatter(idx_c, [pos], idx, mask=m)
                plsc.store_scatter(dloc_c, [pos], dv - lo, mask=m)
                ptr = ptr + plsc.all_reduce_population_count(m)[0]
            return ptr
        return lax.fori_loop(0, PCH, _row, ptr)

    _stage(0, bufs_a, sem_sa)
    _stage(1, bufs_b, sem_sb)
    def _pair(i, ptr):
        _stage_wait(bufs_a, sem_sa)
        ptr = _scan(bufs_a, ptr)
        @pl.when(2 * i + 2 < NPH)
        def _():
            _stage(2 * i + 2, bufs_a, sem_sa)
        _stage_wait(bufs_b, sem_sb)
        ptr = _scan(bufs_b, ptr)
        @pl.when(2 * i + 3 < NPH)
        def _():
            _stage(2 * i + 3, bufs_b, sem_sb)
        return ptr
    cnt = lax.fori_loop(0, NPH // 2, _pair, 0)
    _stage_wait(bufs_a, sem_sa)
    cnt = _scan(bufs_a, cnt)

    zi = jnp.zeros((16,), jnp.int32)
    di = jnp.full((16,), NPT, jnp.int32)
    lane = lax.broadcasted_iota(jnp.int32, (16,), 0)
    def _pad(k, c):
        pos = jnp.broadcast_to(cnt + k * 16, (16,)) + lane
        plsc.store_scatter(idx_c, [pos], zi)
        plsc.store_scatter(dloc_c, [pos], di)
        return c
    lax.fori_loop(0, 16, _pad, 0)
    nch = 2 * ((cnt + 255) // 256)

    def _g(c, rows, sem):
        pltpu.async_copy(y_hbm.at[idx_c.at[pl.ds(c * GC, GC)]], rows, sem)

    def _gw(rows, sem):
        pltpu.make_async_copy(y_hbm.at[idx_c.at[pl.ds(0, GC)]], rows, sem).wait()

    def _acc(c, rows):
        cb = c * GC
        def _grp(g, cc):
            dv = dloc_c[pl.ds(cb + g * 16, 16)]
            for l in range(16):
                dloc = dv[l]
                j = g * 16 + l
                for k in range(8):
                    sl = pl.ds(k * 16, 16)
                    plsc.addupdate(h2d.at[dloc, sl], rows.at[j][sl])
            return cc
        lax.fori_loop(0, GC // 16, _grp, 0)

    @pl.when(nch > 0)
    def _():
        _g(0, rows_a, sem_a)
    @pl.when(nch > 1)
    def _():
        _g(1, rows_b, sem_b)
    def _gpair(i, c):
        c0 = 2 * i
        _gw(rows_a, sem_a)
        _acc(c0, rows_a)
        @pl.when(c0 + 2 < nch)
        def _():
            _g(c0 + 2, rows_a, sem_a)
        _gw(rows_b, sem_b)
        _acc(c0 + 1, rows_b)
        @pl.when(c0 + 3 < nch)
        def _():
            _g(c0 + 3, rows_b, sem_b)
        return c
    lax.fori_loop(0, nch // 2, _gpair, 0)

    @pl.when(wid < NW - 1)
    def _():
        pltpu.sync_copy(h2d.at[pl.ds(0, NPT)], out_hbm.at[pl.ds(lo, NPT)])
    @pl.when(wid == NW - 1)
    def _():
        pltpu.sync_copy(h2d.at[pl.ds(0, NLAST)], out_hbm.at[pl.ds(lo, NLAST)])


def _make_sc(y, src_r, dst_r, typ_r):
    mesh = plsc.VectorSubcoreMesh(core_axis_name="c", subcore_axis_name="s")
    f = pl.kernel(
        _sc_body,
        out_type=jax.ShapeDtypeStruct((N, 128), jnp.float32),
        mesh=mesh,
        compiler_params=pltpu.CompilerParams(
            use_tc_tiling_on_sc=False, needs_layout_passes=False),
        scratch_types=[
            pltpu.VMEM((PCH, CS), jnp.int32),
            pltpu.VMEM((PCH, CS), jnp.int32),
            pltpu.VMEM((PCH, CS), jnp.int32),
            pltpu.VMEM((PCH, CS), jnp.int32),
            pltpu.VMEM((PCH, CS), jnp.int32),
            pltpu.VMEM((PCH, CS), jnp.int32),
            pltpu.VMEM((CAP,), jnp.int32),
            pltpu.VMEM((CAP,), jnp.int32),
            pltpu.VMEM((NPT + 1, 128), jnp.float32),
            pltpu.VMEM((GC, 128), jnp.float32),
            pltpu.VMEM((GC, 128), jnp.float32),
            pltpu.SemaphoreType.DMA,
            pltpu.SemaphoreType.DMA,
            pltpu.SemaphoreType.DMA,
            pltpu.SemaphoreType.DMA,
        ],
    )
    return f(y, src_r, dst_r, typ_r)


def _fin_body(h_ref, x_ref, norm_ref, lw_ref, b_ref, o_ref):
    lm = jnp.dot(x_ref[...], lw_ref[...], preferred_element_type=jnp.float32)
    o_ref[...] = h_ref[...] * norm_ref[...] + b_ref[...] + lm


def _make_fin(h, x, norm, loop_weight, bias2):
    return pl.pallas_call(
        _fin_body,
        grid=(NT,),
        in_specs=[
            pl.BlockSpec((TN, 128), lambda i: (i, 0)),
            pl.BlockSpec((TN, 128), lambda i: (i, 0)),
            pl.BlockSpec((TN, 1), lambda i: (i, 0)),
            pl.BlockSpec((128, 128), lambda i: (0, 0)),
            pl.BlockSpec((1, 128), lambda i: (0, 0)),
        ],
        out_specs=pl.BlockSpec((TN, 128), lambda i: (i, 0)),
        out_shape=jax.ShapeDtypeStruct((N, 128), jnp.float32),
    )(h, x, norm, loop_weight, bias2)


def kernel(x, edge_index, edge_type, norm, weight, loop_weight, bias_parm):
    wr = weight.reshape(NUM_RELS * 128, 32)
    src_r = edge_index[0].reshape(NPH, PCH, CS)
    dst_r = edge_index[1].reshape(NPH, PCH, CS)
    typ_r = edge_type.reshape(NPH, PCH, CS)

    y = _make_y(x, wr)
    h = _make_sc(y, src_r, dst_r, typ_r)
    return _make_fin(h, x, norm, loop_weight, bias_parm.reshape(1, 128))
